# baseline (speedup 1.0000x reference)
import functools

import jax
import jax.numpy as jnp
from jax import lax
from jax.experimental import pallas as pl
from jax.experimental.pallas import tpu as pltpu
from jax.experimental.pallas import tpu_sc as plsc

N_CLUSTERS = 4
KMEANS_ITERS = 10
_HIGH = lax.Precision.HIGHEST
_IPB = 16


def _stageA_body(x_ref, o_ref):
    x = x_ref[...]
    xs = jnp.sum(x.reshape(_IPB, 3, 14, 16, 224), axis=2)
    xm = xs.reshape(_IPB * 48, 224)
    w_idx = lax.broadcasted_iota(jnp.int32, (224, 16), 0) % 16
    j_idx = lax.broadcasted_iota(jnp.int32, (224, 16), 1)
    S = (w_idx == j_idx).astype(jnp.float32)
    y = jnp.dot(xm, S, preferred_element_type=jnp.float32, precision=_HIGH)
    o_ref[...] = y.reshape(_IPB, 48, 16) * (1.0 / 196.0)


def _stageB_body(g_ref, wp_ref, bp_ref, f_ref, gb_ref):
    g = g_ref[...]
    f = jnp.dot(g, wp_ref[...], preferred_element_type=jnp.float32,
                precision=_HIGH) + bp_ref[...]
    f_ref[...] = f

    nrm = jnp.sqrt(jnp.sum(f * f, axis=1, keepdims=True))
    xn = f / (nrm + 1e-8)

    r_i = lax.broadcasted_iota(jnp.int32, (128, 96), 0)
    c_i = lax.broadcasted_iota(jnp.int32, (128, 96), 1)
    E = ((c_i == 12 * (r_i // 16) + r_i % 16) & (r_i % 16 < 12)).astype(
        jnp.float32)
    xe = jnp.dot(E, xn, preferred_element_type=jnp.float32,
                 precision=_HIGH)

    blocks = []
    for s in range(8):
        b = xe[16 * s:16 * s + 16, :]
        blocks.append(lax.dot_general(
            b, b, (((1,), (1,)), ((), ())),
            preferred_element_type=jnp.float32, precision=_HIGH))
    gb_ref[...] = jnp.concatenate(blocks, axis=0)


def _gather16(x, idx):
    dnums = lax.GatherDimensionNumbers(
        offset_dims=(), collapsed_slice_dims=(0,), start_index_map=(0,))
    return lax.gather(x, idx[:, None], dnums, (1,),
                      mode=lax.GatherScatterMode.PROMISE_IN_BOUNDS)


def _rsqrt16(q):
    i = lax.bitcast_convert_type(q, jnp.int32)
    i = 0x5F3759DF - lax.shift_right_logical(i, 1)
    x = lax.bitcast_convert_type(i, jnp.float32)
    for _ in range(3):
        x = x * (1.5 - 0.5 * q * x * x)
    return x


def _allsum16(x, lane):
    for sh in (1, 2, 4, 8):
        x = x + _gather16(x, lane ^ sh)
    return x


def _stageC_body(gb_hbm, f_hbm, out_hbm, g_v, f_v, o_v):
    w = lax.axis_index("s") * 2 + lax.axis_index("c")
    s = w // 4

    pltpu.sync_copy(gb_hbm.at[s], g_v)
    pltpu.sync_copy(f_hbm.at[w], f_v)

    gcol = [g_v[j] for j in range(12)]
    lane = lax.broadcasted_iota(jnp.int32, (16,), 0)
    valid = lane < 12

    def assign(ws):
        sims = []
        for k in range(N_CLUSTERS):
            dot = jnp.zeros((16,), jnp.float32)
            for j in range(12):
                wj = _gather16(ws[k], jnp.full((16,), j, jnp.int32))
                dot = dot + wj * gcol[j]
            q = _allsum16(ws[k] * dot, lane)
            q = jnp.maximum(q, 1e-30)
            nrm = q * _rsqrt16(q)
            sims.append(dot / (nrm + 1e-8))
        best = sims[0]
        bk = jnp.zeros((16,), jnp.int32)
        for k in range(1, N_CLUSTERS):
            gt = sims[k] > best
            best = jnp.where(gt, sims[k], best)
            bk = jnp.where(gt, jnp.full((16,), k, jnp.int32), bk)
        return bk

    def body(_, ws):
        bk = assign(ws)
        new_ws = []
        for k in range(N_CLUSTERS):
            oh = (bk == k) & valid
            ohf = jnp.where(oh, 1.0, 0.0)
            cnt = _allsum16(ohf, lane)
            cf = jnp.maximum(cnt, 1.0)
            new_ws.append(jnp.where(cnt > 0.5, ohf / cf, ws[k]))
        return tuple(new_ws)

    onev = jnp.full((16,), 1.0, jnp.float32)
    zerov = jnp.zeros((16,), jnp.float32)
    ws0 = tuple(jnp.where(lane == k, onev, zerov) for k in range(N_CLUSTERS))
    ws = lax.fori_loop(0, KMEANS_ITERS, body, ws0)
    bk = assign(ws)

    acc = [[jnp.zeros((16,), jnp.float32) for _ in range(12)]
           for _ in range(N_CLUSTERS)]
    for i in range(12):
        labi = _gather16(bk, jnp.full((16,), i, jnp.int32))
        row = [f_v[i, pl.ds(16 * v, 16)] for v in range(12)]
        for k in range(N_CLUSTERS):
            mk = jnp.where(labi == k, 1.0, 0.0)
            for v in range(12):
                acc[k][v] = acc[k][v] + row[v] * mk
    for v in range(12):
        m01 = jnp.maximum(acc[0][v], acc[1][v])
        m23 = jnp.maximum(acc[2][v], acc[3][v])
        o_v[pl.ds(16 * v, 16)] = jnp.maximum(m01, m23)

    pltpu.sync_copy(o_v, out_hbm.at[w])


_stageC = functools.partial(
    pl.kernel,
    out_type=jax.ShapeDtypeStruct((32, 192), jnp.float32),
    mesh=plsc.VectorSubcoreMesh(core_axis_name="c", subcore_axis_name="s"),
    scratch_types=[
        pltpu.VMEM((16, 16), jnp.float32),
        pltpu.VMEM((12, 192), jnp.float32),
        pltpu.VMEM((192,), jnp.float32),
    ],
)(_stageC_body)


@jax.jit
def kernel(mvimages, W_patch, b_patch):
    B, M, C, H, W = mvimages.shape
    N = B * M
    x4 = mvimages.reshape(N, C, H, W)

    g = pl.pallas_call(
        _stageA_body,
        grid=(N // _IPB,),
        in_specs=[pl.BlockSpec((_IPB, C, H, W), lambda n: (n, 0, 0, 0))],
        out_specs=pl.BlockSpec((_IPB, 48, 16), lambda n: (n, 0, 0)),
        out_shape=jax.ShapeDtypeStruct((N, 48, 16), jnp.float32),
    )(x4)

    g2 = g.reshape(N, 768)
    bp2 = b_patch.reshape(1, 768)

    f, gb = pl.pallas_call(
        _stageB_body,
        in_specs=[
            pl.BlockSpec((N, 768), lambda: (0, 0)),
            pl.BlockSpec((768, 768), lambda: (0, 0)),
            pl.BlockSpec((1, 768), lambda: (0, 0)),
        ],
        out_specs=[
            pl.BlockSpec((N, 768), lambda: (0, 0)),
            pl.BlockSpec((128, 16), lambda: (0, 0)),
        ],
        out_shape=[
            jax.ShapeDtypeStruct((N, 768), jnp.float32),
            jax.ShapeDtypeStruct((128, 16), jnp.float32),
        ],
    )(g2, W_patch, bp2)

    f_sc = f.reshape(B, M, 4, 192).transpose(0, 2, 1, 3).reshape(32, M, 192)
    gb3 = gb.reshape(8, 16, 16)
    out = _stageC(gb3, f_sc)
    return out.reshape(B, 768)


# speedup vs baseline: 1.0419x; 1.0419x over previous
import functools

import jax
import jax.numpy as jnp
from jax import lax
from jax.experimental import pallas as pl
from jax.experimental.pallas import tpu as pltpu
from jax.experimental.pallas import tpu_sc as plsc

N_CLUSTERS = 4
KMEANS_ITERS = 10
_HIGH = lax.Precision.DEFAULT
_IPB = 16


def _stageA_body(x_ref, o_ref):
    x = x_ref[...]
    xs = jnp.sum(x.reshape(_IPB, 3, 14, 16, 224), axis=2)
    xm = xs.reshape(_IPB * 48, 224)
    w_idx = lax.broadcasted_iota(jnp.int32, (224, 16), 0) % 16
    j_idx = lax.broadcasted_iota(jnp.int32, (224, 16), 1)
    S = (w_idx == j_idx).astype(jnp.float32)
    y = jnp.dot(xm, S, preferred_element_type=jnp.float32, precision=_HIGH)
    o_ref[...] = y.reshape(_IPB, 48, 16) * (1.0 / 196.0)


def _stageB_body(g_ref, wp_ref, bp_ref, f_ref, gb_ref):
    g = g_ref[...]
    f = jnp.dot(g, wp_ref[...], preferred_element_type=jnp.float32,
                precision=_HIGH) + bp_ref[...]
    f_ref[...] = f

    nrm = jnp.sqrt(jnp.sum(f * f, axis=1, keepdims=True))
    xn = f / (nrm + 1e-8)

    r_i = lax.broadcasted_iota(jnp.int32, (128, 96), 0)
    c_i = lax.broadcasted_iota(jnp.int32, (128, 96), 1)
    E = ((c_i == 12 * (r_i // 16) + r_i % 16) & (r_i % 16 < 12)).astype(
        jnp.float32)
    xe = jnp.dot(E, xn, preferred_element_type=jnp.float32,
                 precision=_HIGH)

    blocks = []
    for s in range(8):
        b = xe[16 * s:16 * s + 16, :]
        blocks.append(lax.dot_general(
            b, b, (((1,), (1,)), ((), ())),
            preferred_element_type=jnp.float32, precision=_HIGH))
    gb_ref[...] = jnp.concatenate(blocks, axis=0)


def _gather16(x, idx):
    dnums = lax.GatherDimensionNumbers(
        offset_dims=(), collapsed_slice_dims=(0,), start_index_map=(0,))
    return lax.gather(x, idx[:, None], dnums, (1,),
                      mode=lax.GatherScatterMode.PROMISE_IN_BOUNDS)


def _rsqrt16(q):
    i = lax.bitcast_convert_type(q, jnp.int32)
    i = 0x5F3759DF - lax.shift_right_logical(i, 1)
    x = lax.bitcast_convert_type(i, jnp.float32)
    for _ in range(3):
        x = x * (1.5 - 0.5 * q * x * x)
    return x


def _allsum16(x, lane):
    for sh in (1, 2, 4, 8):
        x = x + _gather16(x, lane ^ sh)
    return x


def _stageC_body(gb_hbm, f_hbm, out_hbm, g_v, f_v, o_v):
    w = lax.axis_index("s") * 2 + lax.axis_index("c")
    s = w // 4

    pltpu.sync_copy(gb_hbm.at[s], g_v)
    pltpu.sync_copy(f_hbm.at[w], f_v)

    gcol = [g_v[j] for j in range(12)]
    lane = lax.broadcasted_iota(jnp.int32, (16,), 0)
    valid = lane < 12

    def assign(ws):
        sims = []
        for k in range(N_CLUSTERS):
            dot = jnp.zeros((16,), jnp.float32)
            for j in range(12):
                wj = _gather16(ws[k], jnp.full((16,), j, jnp.int32))
                dot = dot + wj * gcol[j]
            q = _allsum16(ws[k] * dot, lane)
            q = jnp.maximum(q, 1e-30)
            nrm = q * _rsqrt16(q)
            sims.append(dot / (nrm + 1e-8))
        best = sims[0]
        bk = jnp.zeros((16,), jnp.int32)
        for k in range(1, N_CLUSTERS):
            gt = sims[k] > best
            best = jnp.where(gt, sims[k], best)
            bk = jnp.where(gt, jnp.full((16,), k, jnp.int32), bk)
        return bk

    def body(_, ws):
        bk = assign(ws)
        new_ws = []
        for k in range(N_CLUSTERS):
            oh = (bk == k) & valid
            ohf = jnp.where(oh, 1.0, 0.0)
            cnt = _allsum16(ohf, lane)
            cf = jnp.maximum(cnt, 1.0)
            new_ws.append(jnp.where(cnt > 0.5, ohf / cf, ws[k]))
        return tuple(new_ws)

    onev = jnp.full((16,), 1.0, jnp.float32)
    zerov = jnp.zeros((16,), jnp.float32)
    ws0 = tuple(jnp.where(lane == k, onev, zerov) for k in range(N_CLUSTERS))
    ws = lax.fori_loop(0, KMEANS_ITERS, body, ws0)
    bk = assign(ws)

    acc = [[jnp.zeros((16,), jnp.float32) for _ in range(12)]
           for _ in range(N_CLUSTERS)]
    for i in range(12):
        labi = _gather16(bk, jnp.full((16,), i, jnp.int32))
        row = [f_v[i, pl.ds(16 * v, 16)] for v in range(12)]
        for k in range(N_CLUSTERS):
            mk = jnp.where(labi == k, 1.0, 0.0)
            for v in range(12):
                acc[k][v] = acc[k][v] + row[v] * mk
    for v in range(12):
        m01 = jnp.maximum(acc[0][v], acc[1][v])
        m23 = jnp.maximum(acc[2][v], acc[3][v])
        o_v[pl.ds(16 * v, 16)] = jnp.maximum(m01, m23)

    pltpu.sync_copy(o_v, out_hbm.at[w])


_stageC = functools.partial(
    pl.kernel,
    out_type=jax.ShapeDtypeStruct((32, 192), jnp.float32),
    mesh=plsc.VectorSubcoreMesh(core_axis_name="c", subcore_axis_name="s"),
    scratch_types=[
        pltpu.VMEM((16, 16), jnp.float32),
        pltpu.VMEM((12, 192), jnp.float32),
        pltpu.VMEM((192,), jnp.float32),
    ],
)(_stageC_body)


@jax.jit
def kernel(mvimages, W_patch, b_patch):
    B, M, C, H, W = mvimages.shape
    N = B * M
    x4 = mvimages.reshape(N, C, H, W)

    g = pl.pallas_call(
        _stageA_body,
        grid=(N // _IPB,),
        in_specs=[pl.BlockSpec((_IPB, C, H, W), lambda n: (n, 0, 0, 0))],
        out_specs=pl.BlockSpec((_IPB, 48, 16), lambda n: (n, 0, 0)),
        out_shape=jax.ShapeDtypeStruct((N, 48, 16), jnp.float32),
    )(x4)

    g2 = g.reshape(N, 768)
    bp2 = b_patch.reshape(1, 768)

    f, gb = pl.pallas_call(
        _stageB_body,
        in_specs=[
            pl.BlockSpec((N, 768), lambda: (0, 0)),
            pl.BlockSpec((768, 768), lambda: (0, 0)),
            pl.BlockSpec((1, 768), lambda: (0, 0)),
        ],
        out_specs=[
            pl.BlockSpec((N, 768), lambda: (0, 0)),
            pl.BlockSpec((128, 16), lambda: (0, 0)),
        ],
        out_shape=[
            jax.ShapeDtypeStruct((N, 768), jnp.float32),
            jax.ShapeDtypeStruct((128, 16), jnp.float32),
        ],
    )(g2, W_patch, bp2)

    f_sc = f.reshape(B, M, 4, 192).transpose(0, 2, 1, 3).reshape(32, M, 192)
    gb3 = gb.reshape(8, 16, 16)
    out = _stageC(gb3, f_sc)
    return out.reshape(B, 768)
